# baseline (speedup 1.0000x reference)
import functools

import jax
import jax.numpy as jnp
import numpy as np
from jax.experimental import pallas as pl
from jax.experimental.pallas import tpu as pltpu

D_MODEL = 768
NUM_HEADS = 12
HEAD_DIM = 64
BS = 128
TOP_K = 3
CHUNK = 512
BPC = CHUNK // BS
HPG = 12
NGROUPS = NUM_HEADS // HPG

NEG_INF = float("-inf")


def _head_attention(q_t, k_t, v_t, seq_len):
    nb = seq_len // BS
    scale = 1.0 / np.sqrt(HEAD_DIM)

    k_mean_t = jnp.mean(k_t.reshape(HEAD_DIM, nb, BS), axis=2)
    gate = jax.lax.dot_general(
        k_mean_t, q_t, (((0,), (0,)), ((), ())),
        preferred_element_type=jnp.float32)
    blk = jax.lax.broadcasted_iota(jnp.int32, (nb, seq_len), 0)
    qblk = jax.lax.broadcasted_iota(jnp.int32, (nb, seq_len), 1) // BS
    gate = jnp.where(blk > qblk, NEG_INF, gate)

    sel = jnp.zeros((nb, seq_len), jnp.float32)
    g = gate
    for _ in range(TOP_K):
        m = jnp.max(g, axis=0, keepdims=True)
        is_max = g == m
        first_idx = jnp.min(jnp.where(is_max, blk, nb), axis=0,
                            keepdims=True)
        pick = blk == first_idx
        sel = jnp.maximum(sel, pick.astype(jnp.float32))
        g = jnp.where(pick, NEG_INF, g)
    w_t = sel * (blk < qblk).astype(jnp.float32)

    rr = jax.lax.broadcasted_iota(jnp.int32, (BS, BS), 0)
    cc = jax.lax.broadcasted_iota(jnp.int32, (BS, BS), 1)
    causal_f = (rr <= cc).astype(jnp.float32)
    self_outs = []
    for i in range(nb):
        q_i = q_t[:, i * BS:(i + 1) * BS]
        k_i = k_t[:, i * BS:(i + 1) * BS]
        v_i = v_t[:, i * BS:(i + 1) * BS]
        s_t = jax.lax.dot_general(
            k_i, q_i, (((0,), (0,)), ((), ())),
            preferred_element_type=jnp.float32) * scale
        e_t = jnp.exp(s_t) * causal_f
        den_t = jnp.sum(e_t, axis=0, keepdims=True)
        num_t = jax.lax.dot_general(
            v_i, e_t, (((1,), (0,)), ((), ())),
            preferred_element_type=jnp.float32)
        self_outs.append(num_t / den_t)

    dr = jax.lax.broadcasted_iota(jnp.int32, (CHUNK, BPC), 0)
    dc = jax.lax.broadcasted_iota(jnp.int32, (CHUNK, BPC), 1)
    dmat = (dr // BS == dc).astype(jnp.float32)

    adds = []
    for cidx in range(seq_len // CHUNK):
        col0 = (cidx * BPC + 1) * BS
        q_c = q_t[:, col0:]
        k_c = k_t[:, cidx * CHUNK:(cidx + 1) * CHUNK]
        s_t = jax.lax.dot_general(
            k_c, q_c, (((0,), (0,)), ((), ())),
            preferred_element_type=jnp.float32) * scale
        e_t = jnp.exp(s_t)
        den_t = jax.lax.dot_general(
            dmat, e_t, (((0,), (0,)), ((), ())),
            preferred_element_type=jnp.float32)
        coef_t = w_t[cidx * BPC:(cidx + 1) * BPC, col0:] / den_t
        acc_t = None
        for b in range(BPC):
            num_t = jax.lax.dot_general(
                v_t[:, cidx * CHUNK + b * BS:cidx * CHUNK + (b + 1) * BS],
                e_t[b * BS:(b + 1) * BS, :], (((1,), (0,)), ((), ())),
                preferred_element_type=jnp.float32)
            contrib = num_t * coef_t[b:b + 1, :]
            acc_t = contrib if acc_t is None else acc_t + contrib
        adds.append((col0, acc_t))
    for col0, acc_t in adds:
        for i in range(col0 // BS, nb):
            self_outs[i] = self_outs[i] + acc_t[:, i * BS - col0:(i + 1) * BS - col0]
    return jnp.concatenate(self_outs, axis=1)


def _fused_body(x_ref, wq_ref, bq_ref, wk_ref, bk_ref, wv_ref, bv_ref,
                wo_ref, bo_ref, o_ref, scr_ref, *, seq_len):
    g = pl.program_id(0)
    xv = x_ref[:]
    dn = (((1,), (1,)), ((), ()))
    qg_t = jax.lax.dot_general(
        wq_ref[:], xv, dn, preferred_element_type=jnp.float32) + bq_ref[:]
    kg_t = jax.lax.dot_general(
        wk_ref[:], xv, dn, preferred_element_type=jnp.float32) + bk_ref[:]
    vg_t = jax.lax.dot_general(
        wv_ref[:], xv, dn, preferred_element_type=jnp.float32) + bv_ref[:]

    outs = []
    for hl in range(HPG):
        q_t = qg_t[hl * HEAD_DIM:(hl + 1) * HEAD_DIM, :]
        k_t = kg_t[hl * HEAD_DIM:(hl + 1) * HEAD_DIM, :]
        v_t = vg_t[hl * HEAD_DIM:(hl + 1) * HEAD_DIM, :]
        outs.append(_head_attention(q_t, k_t, v_t, seq_len))
    attn_g_t = jnp.concatenate(outs, axis=0)
    gw = HPG * HEAD_DIM

    @pl.when(g < NGROUPS - 1)
    def _():
        scr_ref[pl.ds(g * gw, gw), :] = attn_g_t

    @pl.when(g == NGROUPS - 1)
    def _():
        parts = [scr_ref[gg * gw:(gg + 1) * gw, :]
                 for gg in range(NGROUPS - 1)]
        full_t = jnp.concatenate(parts + [attn_g_t], axis=0)
        o_ref[:] = jax.lax.dot_general(
            full_t, wo_ref[:], (((0,), (1,)), ((), ())),
            preferred_element_type=jnp.float32) + bo_ref[:]


def kernel(x, Wq, bq, Wk, bk, Wv, bv, Wo, bo):
    Bc, S, D = x.shape
    x2 = x.reshape(S, D)
    gw = HPG * HEAD_DIM

    wspec = pl.BlockSpec((gw, D), lambda g: (g, 0))
    bspec = pl.BlockSpec((gw, 1), lambda g: (g, 0))
    xspec = pl.BlockSpec((S, D), lambda g: (0, 0))
    wospec = pl.BlockSpec((D, D), lambda g: (0, 0))
    bospec = pl.BlockSpec((1, D), lambda g: (0, 0))

    y = pl.pallas_call(
        functools.partial(_fused_body, seq_len=S),
        grid=(NGROUPS,),
        in_specs=[xspec, wspec, bspec, wspec, bspec, wspec, bspec,
                  wospec, bospec],
        out_specs=pl.BlockSpec((S, D), lambda g: (0, 0)),
        out_shape=jax.ShapeDtypeStruct((S, D), jnp.float32),
        scratch_shapes=[pltpu.VMEM((D, S), jnp.float32)],
    )(x2, Wq, bq.reshape(D, 1), Wk, bk.reshape(D, 1),
      Wv, bv.reshape(D, 1), Wo, bo.reshape(1, D))
    return y.reshape(Bc, S, D)


# speedup vs baseline: 1.0176x; 1.0176x over previous
import functools

import jax
import jax.numpy as jnp
import numpy as np
from jax.experimental import pallas as pl
from jax.experimental.pallas import tpu as pltpu

D_MODEL = 768
NUM_HEADS = 12
HEAD_DIM = 64
BS = 128
TOP_K = 3
CHUNK = 512
BPC = CHUNK // BS
HPG = 6
NGROUPS = NUM_HEADS // HPG

NEG_INF = float("-inf")


def _head_attention(q_t, k_t, v_t, seq_len):
    nb = seq_len // BS
    scale = 1.0 / np.sqrt(HEAD_DIM)

    k_mean_t = jnp.mean(k_t.reshape(HEAD_DIM, nb, BS), axis=2)
    gate = jax.lax.dot_general(
        k_mean_t, q_t, (((0,), (0,)), ((), ())),
        preferred_element_type=jnp.float32)
    blk = jax.lax.broadcasted_iota(jnp.int32, (nb, seq_len), 0)
    qblk = jax.lax.broadcasted_iota(jnp.int32, (nb, seq_len), 1) // BS
    gate = jnp.where(blk > qblk, NEG_INF, gate)

    sel = jnp.zeros((nb, seq_len), jnp.float32)
    g = gate
    for _ in range(TOP_K):
        m = jnp.max(g, axis=0, keepdims=True)
        is_max = g == m
        first_idx = jnp.min(jnp.where(is_max, blk, nb), axis=0,
                            keepdims=True)
        pick = blk == first_idx
        sel = jnp.maximum(sel, pick.astype(jnp.float32))
        g = jnp.where(pick, NEG_INF, g)
    w_t = sel * (blk < qblk).astype(jnp.float32)

    rr = jax.lax.broadcasted_iota(jnp.int32, (BS, BS), 0)
    cc = jax.lax.broadcasted_iota(jnp.int32, (BS, BS), 1)
    causal_f = (rr <= cc).astype(jnp.float32)
    self_outs = []
    for i in range(nb):
        q_i = q_t[:, i * BS:(i + 1) * BS]
        k_i = k_t[:, i * BS:(i + 1) * BS]
        v_i = v_t[:, i * BS:(i + 1) * BS]
        s_t = jax.lax.dot_general(
            k_i, q_i, (((0,), (0,)), ((), ())),
            preferred_element_type=jnp.float32) * scale
        e_t = jnp.exp(s_t) * causal_f
        den_t = jnp.sum(e_t, axis=0, keepdims=True)
        num_t = jax.lax.dot_general(
            v_i, e_t, (((1,), (0,)), ((), ())),
            preferred_element_type=jnp.float32)
        self_outs.append(num_t / den_t)

    dr = jax.lax.broadcasted_iota(jnp.int32, (CHUNK, BPC), 0)
    dc = jax.lax.broadcasted_iota(jnp.int32, (CHUNK, BPC), 1)
    dmat = (dr // BS == dc).astype(jnp.float32)

    adds = []
    for cidx in range(seq_len // CHUNK):
        col0 = (cidx * BPC + 1) * BS
        q_c = q_t[:, col0:]
        k_c = k_t[:, cidx * CHUNK:(cidx + 1) * CHUNK]
        s_t = jax.lax.dot_general(
            k_c, q_c, (((0,), (0,)), ((), ())),
            preferred_element_type=jnp.float32) * scale
        e_t = jnp.exp(s_t)
        den_t = jax.lax.dot_general(
            dmat, e_t, (((0,), (0,)), ((), ())),
            preferred_element_type=jnp.float32)
        coef_t = w_t[cidx * BPC:(cidx + 1) * BPC, col0:] / den_t
        acc_t = None
        for b in range(BPC):
            num_t = jax.lax.dot_general(
                v_t[:, cidx * CHUNK + b * BS:cidx * CHUNK + (b + 1) * BS],
                e_t[b * BS:(b + 1) * BS, :], (((1,), (0,)), ((), ())),
                preferred_element_type=jnp.float32)
            contrib = num_t * coef_t[b:b + 1, :]
            acc_t = contrib if acc_t is None else acc_t + contrib
        adds.append((col0, acc_t))
    for col0, acc_t in adds:
        for i in range(col0 // BS, nb):
            self_outs[i] = self_outs[i] + acc_t[:, i * BS - col0:(i + 1) * BS - col0]
    return jnp.concatenate(self_outs, axis=1)


def _fused_body(x_ref, wq_ref, bq_ref, wk_ref, bk_ref, wv_ref, bv_ref,
                wo_ref, bo_ref, o_ref, scr_ref, *, seq_len):
    g = pl.program_id(0)
    xv = x_ref[:]
    dn = (((1,), (1,)), ((), ()))
    qg_t = jax.lax.dot_general(
        wq_ref[:], xv, dn, preferred_element_type=jnp.float32) + bq_ref[:]
    kg_t = jax.lax.dot_general(
        wk_ref[:], xv, dn, preferred_element_type=jnp.float32) + bk_ref[:]
    vg_t = jax.lax.dot_general(
        wv_ref[:], xv, dn, preferred_element_type=jnp.float32) + bv_ref[:]

    outs = []
    for hl in range(HPG):
        q_t = qg_t[hl * HEAD_DIM:(hl + 1) * HEAD_DIM, :]
        k_t = kg_t[hl * HEAD_DIM:(hl + 1) * HEAD_DIM, :]
        v_t = vg_t[hl * HEAD_DIM:(hl + 1) * HEAD_DIM, :]
        outs.append(_head_attention(q_t, k_t, v_t, seq_len))
    attn_g_t = jnp.concatenate(outs, axis=0)
    gw = HPG * HEAD_DIM

    @pl.when(g < NGROUPS - 1)
    def _():
        scr_ref[pl.ds(g * gw, gw), :] = attn_g_t

    @pl.when(g == NGROUPS - 1)
    def _():
        parts = [scr_ref[gg * gw:(gg + 1) * gw, :]
                 for gg in range(NGROUPS - 1)]
        full_t = jnp.concatenate(parts + [attn_g_t], axis=0)
        o_ref[:] = jax.lax.dot_general(
            full_t, wo_ref[:], (((0,), (1,)), ((), ())),
            preferred_element_type=jnp.float32) + bo_ref[:]


def kernel(x, Wq, bq, Wk, bk, Wv, bv, Wo, bo):
    Bc, S, D = x.shape
    x2 = x.reshape(S, D)
    gw = HPG * HEAD_DIM

    wspec = pl.BlockSpec((gw, D), lambda g: (g, 0))
    bspec = pl.BlockSpec((gw, 1), lambda g: (g, 0))
    xspec = pl.BlockSpec((S, D), lambda g: (0, 0))
    wospec = pl.BlockSpec((D, D), lambda g: (0, 0))
    bospec = pl.BlockSpec((1, D), lambda g: (0, 0))

    y = pl.pallas_call(
        functools.partial(_fused_body, seq_len=S),
        grid=(NGROUPS,),
        in_specs=[xspec, wspec, bspec, wspec, bspec, wspec, bspec,
                  wospec, bospec],
        out_specs=pl.BlockSpec((S, D), lambda g: (0, 0)),
        out_shape=jax.ShapeDtypeStruct((S, D), jnp.float32),
        scratch_shapes=[pltpu.VMEM((D, S), jnp.float32)],
    )(x2, Wq, bq.reshape(D, 1), Wk, bk.reshape(D, 1),
      Wv, bv.reshape(D, 1), Wo, bo.reshape(1, D))
    return y.reshape(Bc, S, D)


# speedup vs baseline: 1.0271x; 1.0094x over previous
import functools

import jax
import jax.numpy as jnp
import numpy as np
from jax.experimental import pallas as pl
from jax.experimental.pallas import tpu as pltpu

D_MODEL = 768
NUM_HEADS = 12
HEAD_DIM = 64
BS = 128
TOP_K = 3
CHUNK = 512
BPC = CHUNK // BS
HPG = 4
NGROUPS = NUM_HEADS // HPG

NEG_INF = float("-inf")


def _head_attention(q_t, k_t, v_t, seq_len):
    nb = seq_len // BS
    scale = 1.0 / np.sqrt(HEAD_DIM)

    k_mean_t = jnp.mean(k_t.reshape(HEAD_DIM, nb, BS), axis=2)
    gate = jax.lax.dot_general(
        k_mean_t, q_t, (((0,), (0,)), ((), ())),
        preferred_element_type=jnp.float32)
    blk = jax.lax.broadcasted_iota(jnp.int32, (nb, seq_len), 0)
    qblk = jax.lax.broadcasted_iota(jnp.int32, (nb, seq_len), 1) // BS
    gate = jnp.where(blk > qblk, NEG_INF, gate)

    sel = jnp.zeros((nb, seq_len), jnp.float32)
    g = gate
    for _ in range(TOP_K):
        m = jnp.max(g, axis=0, keepdims=True)
        is_max = g == m
        first_idx = jnp.min(jnp.where(is_max, blk, nb), axis=0,
                            keepdims=True)
        pick = blk == first_idx
        sel = jnp.maximum(sel, pick.astype(jnp.float32))
        g = jnp.where(pick, NEG_INF, g)
    w_t = sel * (blk < qblk).astype(jnp.float32)

    rr = jax.lax.broadcasted_iota(jnp.int32, (BS, BS), 0)
    cc = jax.lax.broadcasted_iota(jnp.int32, (BS, BS), 1)
    causal_f = (rr <= cc).astype(jnp.float32)
    self_outs = []
    for i in range(nb):
        q_i = q_t[:, i * BS:(i + 1) * BS]
        k_i = k_t[:, i * BS:(i + 1) * BS]
        v_i = v_t[:, i * BS:(i + 1) * BS]
        s_t = jax.lax.dot_general(
            k_i, q_i, (((0,), (0,)), ((), ())),
            preferred_element_type=jnp.float32) * scale
        e_t = jnp.exp(s_t) * causal_f
        den_t = jnp.sum(e_t, axis=0, keepdims=True)
        num_t = jax.lax.dot_general(
            v_i, e_t, (((1,), (0,)), ((), ())),
            preferred_element_type=jnp.float32)
        self_outs.append(num_t / den_t)

    dr = jax.lax.broadcasted_iota(jnp.int32, (CHUNK, BPC), 0)
    dc = jax.lax.broadcasted_iota(jnp.int32, (CHUNK, BPC), 1)
    dmat = (dr // BS == dc).astype(jnp.float32)

    adds = []
    for cidx in range(seq_len // CHUNK):
        col0 = (cidx * BPC + 1) * BS
        q_c = q_t[:, col0:]
        k_c = k_t[:, cidx * CHUNK:(cidx + 1) * CHUNK]
        s_t = jax.lax.dot_general(
            k_c, q_c, (((0,), (0,)), ((), ())),
            preferred_element_type=jnp.float32) * scale
        e_t = jnp.exp(s_t)
        den_t = jax.lax.dot_general(
            dmat, e_t, (((0,), (0,)), ((), ())),
            preferred_element_type=jnp.float32)
        coef_t = w_t[cidx * BPC:(cidx + 1) * BPC, col0:] / den_t
        acc_t = None
        for b in range(BPC):
            num_t = jax.lax.dot_general(
                v_t[:, cidx * CHUNK + b * BS:cidx * CHUNK + (b + 1) * BS],
                e_t[b * BS:(b + 1) * BS, :], (((1,), (0,)), ((), ())),
                preferred_element_type=jnp.float32)
            contrib = num_t * coef_t[b:b + 1, :]
            acc_t = contrib if acc_t is None else acc_t + contrib
        adds.append((col0, acc_t))
    for col0, acc_t in adds:
        for i in range(col0 // BS, nb):
            self_outs[i] = self_outs[i] + acc_t[:, i * BS - col0:(i + 1) * BS - col0]
    return jnp.concatenate(self_outs, axis=1)


def _fused_body(x_ref, wq_ref, bq_ref, wk_ref, bk_ref, wv_ref, bv_ref,
                wo_ref, bo_ref, o_ref, scr_ref, *, seq_len):
    g = pl.program_id(0)
    xv = x_ref[:]
    dn = (((1,), (1,)), ((), ()))
    qg_t = jax.lax.dot_general(
        wq_ref[:], xv, dn, preferred_element_type=jnp.float32) + bq_ref[:]
    kg_t = jax.lax.dot_general(
        wk_ref[:], xv, dn, preferred_element_type=jnp.float32) + bk_ref[:]
    vg_t = jax.lax.dot_general(
        wv_ref[:], xv, dn, preferred_element_type=jnp.float32) + bv_ref[:]

    outs = []
    for hl in range(HPG):
        q_t = qg_t[hl * HEAD_DIM:(hl + 1) * HEAD_DIM, :]
        k_t = kg_t[hl * HEAD_DIM:(hl + 1) * HEAD_DIM, :]
        v_t = vg_t[hl * HEAD_DIM:(hl + 1) * HEAD_DIM, :]
        outs.append(_head_attention(q_t, k_t, v_t, seq_len))
    attn_g_t = jnp.concatenate(outs, axis=0)
    gw = HPG * HEAD_DIM

    @pl.when(g < NGROUPS - 1)
    def _():
        scr_ref[pl.ds(g * gw, gw), :] = attn_g_t

    @pl.when(g == NGROUPS - 1)
    def _():
        parts = [scr_ref[gg * gw:(gg + 1) * gw, :]
                 for gg in range(NGROUPS - 1)]
        full_t = jnp.concatenate(parts + [attn_g_t], axis=0)
        o_ref[:] = jax.lax.dot_general(
            full_t, wo_ref[:], (((0,), (1,)), ((), ())),
            preferred_element_type=jnp.float32) + bo_ref[:]


def kernel(x, Wq, bq, Wk, bk, Wv, bv, Wo, bo):
    Bc, S, D = x.shape
    x2 = x.reshape(S, D)
    gw = HPG * HEAD_DIM

    wspec = pl.BlockSpec((gw, D), lambda g: (g, 0))
    bspec = pl.BlockSpec((gw, 1), lambda g: (g, 0))
    xspec = pl.BlockSpec((S, D), lambda g: (0, 0))
    wospec = pl.BlockSpec((D, D), lambda g: (0, 0))
    bospec = pl.BlockSpec((1, D), lambda g: (0, 0))

    y = pl.pallas_call(
        functools.partial(_fused_body, seq_len=S),
        grid=(NGROUPS,),
        in_specs=[xspec, wspec, bspec, wspec, bspec, wspec, bspec,
                  wospec, bospec],
        out_specs=pl.BlockSpec((S, D), lambda g: (0, 0)),
        out_shape=jax.ShapeDtypeStruct((S, D), jnp.float32),
        scratch_shapes=[pltpu.VMEM((D, S), jnp.float32)],
    )(x2, Wq, bq.reshape(D, 1), Wk, bk.reshape(D, 1),
      Wv, bv.reshape(D, 1), Wo, bo.reshape(1, D))
    return y.reshape(Bc, S, D)
